# Initial kernel scaffold; baseline (speedup 1.0000x reference)
import jax
import jax.numpy as jnp
from jax.experimental import pallas as pl


def kernel(x, table):
    raise NotImplementedError("write your pallas kernel here")


# speedup vs baseline: 1.0243x; 1.0243x over previous
import functools

import jax
import jax.numpy as jnp
from jax import lax
from jax.experimental import pallas as pl
from jax.experimental.pallas import tpu as pltpu
from jax.experimental.pallas import tpu_sc as plsc

_VOCAB = 1000
_BATCH = 1024
_CTX = 50
_N = _BATCH * _CTX
_NW = 32
_PER_W = _N // _NW
_CHUNK = 64
_NCHUNK = _PER_W // _CHUNK


@functools.partial(
    pl.kernel,
    mesh=plsc.VectorSubcoreMesh(core_axis_name="c", subcore_axis_name="s"),
    out_type=jax.ShapeDtypeStruct((_N, _VOCAB), jnp.float32),
    scratch_types=[
        pltpu.VMEM((_PER_W,), jnp.int32),
        pltpu.VMEM((2, _CHUNK, _VOCAB), jnp.float32),
        pltpu.SemaphoreType.DMA,
        pltpu.SemaphoreType.DMA,
        pltpu.SemaphoreType.DMA,
        pltpu.SemaphoreType.DMA,
    ],
    compiler_params=pltpu.CompilerParams(use_tc_tiling_on_sc=False),
)
def _gather_rows(x_hbm, table_hbm, out_hbm, idx_v, rows_v, gsem0, gsem1,
                 wsem0, wsem1):
    gsems = (gsem0, gsem1)
    wsems = (wsem0, wsem1)
    wid = lax.axis_index("s") * 2 + lax.axis_index("c")
    base = wid * _PER_W

    pltpu.sync_copy(x_hbm.at[pl.ds(base, _PER_W)], idx_v)

    def gather(c):
        buf = c % 2
        return pltpu.async_copy(
            table_hbm.at[idx_v.at[pl.ds(c * _CHUNK, _CHUNK)]],
            rows_v.at[buf],
            gsems[buf],
        )

    def write(c):
        buf = c % 2
        return pltpu.async_copy(
            rows_v.at[buf],
            out_hbm.at[pl.ds(base + c * _CHUNK, _CHUNK)],
            wsems[buf],
        )

    gathers = [None] * _NCHUNK
    writes = [None] * _NCHUNK
    gathers[0] = gather(0)
    for c in range(_NCHUNK):
        gathers[c].wait()
        writes[c] = write(c)
        if c + 1 < _NCHUNK:
            if c >= 1:
                writes[c - 1].wait()
            gathers[c + 1] = gather(c + 1)
    writes[_NCHUNK - 2].wait()
    writes[_NCHUNK - 1].wait()


def kernel(x, table):
    flat = _gather_rows(x.reshape(_N), table)
    return flat.reshape(_BATCH, _CTX, _VOCAB)


# speedup vs baseline: 1.0289x; 1.0045x over previous
import functools

import jax
import jax.numpy as jnp
from jax import lax
from jax.experimental import pallas as pl
from jax.experimental.pallas import tpu as pltpu
from jax.experimental.pallas import tpu_sc as plsc

_VOCAB = 1000
_BATCH = 1024
_CTX = 50
_NW = 32
_PER_W = _BATCH // _NW


@functools.partial(
    pl.kernel,
    mesh=plsc.VectorSubcoreMesh(core_axis_name="c", subcore_axis_name="s"),
    out_type=jax.ShapeDtypeStruct((_BATCH, _CTX, _VOCAB), jnp.float32),
    scratch_types=[
        pltpu.VMEM((_PER_W, _CTX), jnp.int32),
        pltpu.VMEM((2, _CTX, _VOCAB), jnp.float32),
        pltpu.SemaphoreType.DMA,
        pltpu.SemaphoreType.DMA,
        pltpu.SemaphoreType.DMA,
        pltpu.SemaphoreType.DMA,
    ],
    compiler_params=pltpu.CompilerParams(use_tc_tiling_on_sc=False),
)
def _gather_rows(x_hbm, table_hbm, out_hbm, idx_v, rows_v, gsem0, gsem1,
                 wsem0, wsem1):
    gsems = (gsem0, gsem1)
    wsems = (wsem0, wsem1)
    wid = lax.axis_index("s") * 2 + lax.axis_index("c")
    base = wid * _PER_W

    pltpu.sync_copy(x_hbm.at[pl.ds(base, _PER_W)], idx_v)

    def gather(i):
        return pltpu.async_copy(
            table_hbm.at[idx_v.at[i]],
            rows_v.at[i % 2],
            gsems[i % 2],
        )

    def write(i):
        return pltpu.async_copy(
            rows_v.at[i % 2],
            out_hbm.at[base + i],
            wsems[i % 2],
        )

    gathers = [None] * _PER_W
    writes = [None] * _PER_W
    gathers[0] = gather(0)
    for i in range(_PER_W):
        gathers[i].wait()
        writes[i] = write(i)
        if i + 1 < _PER_W:
            if i >= 1:
                writes[i - 1].wait()
            gathers[i + 1] = gather(i + 1)
    writes[_PER_W - 2].wait()
    writes[_PER_W - 1].wait()


def kernel(x, table):
    return _gather_rows(x, table)
